# baseline (speedup 1.0000x reference)
import functools

import jax
import jax.numpy as jnp
from jax import lax
from jax.experimental import pallas as pl
from jax.experimental.pallas import tpu as pltpu
from jax.experimental.pallas import tpu_sc as plsc

N = 10000
D = 128
HID = 16
OUT = 16
TK = 8
E = 320000
H0 = HID * D + HID
H1 = OUT * HID + OUT
NC, NS, CH = 2, 16, 128
GB = 8
GROUPS = 20
G0 = 14
TOT = GROUPS * GB
EPAD = NS * TOT * CH
NPAD = 10240
SLAB = NPAD // NS


def _sc_mesh():
    return plsc.VectorSubcoreMesh(core_axis_name="c", subcore_axis_name="s")


_SC_PARAMS = pltpu.CompilerParams(use_tc_tiling_on_sc=False)


def _deg_body(dst_hbm, zeros_hbm, out_hbm, idx_v, ones_v, deg_sh, sem):
    c = lax.axis_index("c")
    s = lax.axis_index("s")
    for j in range(CH // 16):
        ones_v[pl.ds(j * 16, 16)] = jnp.ones((16,), jnp.float32)
    pltpu.sync_copy(zeros_hbm.at[pl.ds(s * SLAB, SLAB)],
                    deg_sh.at[pl.ds(s * SLAB, SLAB)])
    pltpu.sync_copy(dst_hbm.at[s], idx_v)
    plsc.subcore_barrier()
    start = jnp.where(c == 0, 0, G0 * GB)
    ngroups = jnp.where(c == 0, G0, GROUPS - G0)

    def group(g, carry):
        base = start + g * GB
        cps = [pltpu.async_copy(ones_v, deg_sh.at[idx_v.at[base + b]], sem,
                                add=True)
               for b in range(GB)]
        for cp in cps:
            cp.wait()
        return carry

    lax.fori_loop(0, ngroups, group, 0)
    plsc.subcore_barrier()
    pltpu.sync_copy(deg_sh.at[pl.ds(s * SLAB, SLAB)],
                    out_hbm.at[c, pl.ds(s * SLAB, SLAB)])


def _sc_deg(dst3, zeros1):
    return pl.kernel(
        _deg_body,
        out_type=jax.ShapeDtypeStruct((NC, NPAD), jnp.float32),
        mesh=_sc_mesh(),
        compiler_params=_SC_PARAMS,
        scratch_types=[
            pltpu.VMEM((TOT, CH), jnp.int32),
            pltpu.VMEM((CH,), jnp.float32),
            pltpu.VMEM_SHARED((NPAD,), jnp.float32),
            pltpu.SemaphoreType.DMA,
        ],
    )(dst3, zeros1)


def _agg_body(yw_hbm, src_hbm, dst_hbm, zeros_hbm, out_hbm,
              src_v, dst_v, r_a, r_b, acc_sh, yw_sh, gs_a, gs_b, ss_a, ss_b):
    c = lax.axis_index("c")
    s = lax.axis_index("s")
    pltpu.sync_copy(zeros_hbm.at[pl.ds(s * SLAB, SLAB)],
                    acc_sh.at[pl.ds(s * SLAB, SLAB)])
    pltpu.sync_copy(yw_hbm.at[pl.ds(s * SLAB, SLAB)],
                    yw_sh.at[pl.ds(s * SLAB, SLAB)])
    pltpu.sync_copy(src_hbm.at[s], src_v)
    pltpu.sync_copy(dst_hbm.at[s], dst_v)
    plsc.subcore_barrier()
    start = jnp.where(c == 0, 0, G0 * GB)
    ngroups = jnp.where(c == 0, G0, GROUPS - G0)

    def group2(g, carry):
        base = start + g * (2 * GB)
        ga = [pltpu.async_copy(yw_sh.at[src_v.at[base + b]],
                               r_a.at[pl.ds(b * CH, CH)], gs_a)
              for b in range(GB)]
        gb = [pltpu.async_copy(yw_sh.at[src_v.at[base + GB + b]],
                               r_b.at[pl.ds(b * CH, CH)], gs_b)
              for b in range(GB)]
        for cp in ga:
            cp.wait()
        sa = [pltpu.async_copy(r_a.at[pl.ds(b * CH, CH)],
                               acc_sh.at[dst_v.at[base + b]], ss_a, add=True)
              for b in range(GB)]
        for cp in gb:
            cp.wait()
        sb = [pltpu.async_copy(r_b.at[pl.ds(b * CH, CH)],
                               acc_sh.at[dst_v.at[base + GB + b]], ss_b,
                               add=True)
              for b in range(GB)]
        for cp in sa:
            cp.wait()
        for cp in sb:
            cp.wait()
        return carry

    lax.fori_loop(0, ngroups // 2, group2, 0)
    plsc.subcore_barrier()
    pltpu.sync_copy(acc_sh.at[pl.ds(s * SLAB, SLAB)],
                    out_hbm.at[c, pl.ds(s * SLAB, SLAB)])


def _sc_agg(yw, src3, dst3, zeros2):
    return pl.kernel(
        _agg_body,
        out_type=jax.ShapeDtypeStruct((NC, NPAD, HID), jnp.float32),
        mesh=_sc_mesh(),
        compiler_params=_SC_PARAMS,
        scratch_types=[
            pltpu.VMEM((TOT, CH), jnp.int32),
            pltpu.VMEM((TOT, CH), jnp.int32),
            pltpu.VMEM((GB * CH, HID), jnp.float32),
            pltpu.VMEM((GB * CH, HID), jnp.float32),
            pltpu.VMEM_SHARED((NPAD, HID), jnp.float32),
            pltpu.VMEM_SHARED((NPAD, HID), jnp.float32),
            pltpu.SemaphoreType.DMA,
            pltpu.SemaphoreType.DMA,
            pltpu.SemaphoreType.DMA,
            pltpu.SemaphoreType.DMA,
        ],
    )(yw, src3, dst3, zeros2)


BN = 2048
NB = NPAD // BN
RB = 1032
NGRU = 3 * H0 // RB


def _topk_z(h_s, y, zc_s, nvalid, k, d):
    y2 = y.reshape(NPAD // 128, 128)
    gidx = lax.broadcasted_iota(jnp.int32, y2.shape, 0) * 128 \
        + lax.broadcasted_iota(jnp.int32, y2.shape, 1)
    y2 = jnp.where(gidx < nvalid, y2, -jnp.inf)
    for j in range(k):
        mx = jnp.max(y2)
        idx = jnp.min(jnp.where(y2 == mx, gidx, nvalid))
        w = jnp.tanh(mx)
        row = h_s[pl.ds(idx, 1), :]
        zc_s[pl.ds(j * d, d), :] = (row * w).T
        y2 = jnp.where(gidx == idx, -jnp.inf, y2)


def _gates(gi, gh, bih, bhh, h0, h):
    r = jax.nn.sigmoid(gi[0:h, :] + bih[0:h, :]
                       + gh[0:h, :] + bhh[0:h, :])
    z = jax.nn.sigmoid(gi[h:2 * h, :] + bih[h:2 * h, :]
                       + gh[h:2 * h, :] + bhh[h:2 * h, :])
    n = jnp.tanh(gi[2 * h:3 * h, :] + bih[2 * h:3 * h, :]
                 + r * (gh[2 * h:3 * h, :] + bhh[2 * h:3 * h, :]))
    return (1.0 - z) * n + z * h0


def _dinv_col(degt_ref, j):
    dt = degt_ref[pl.ds(j * BN, BN), :]
    return lax.rsqrt(dt[:, 0] + dt[:, 1] + 1.0)[:, None]


def _ka_body(x_ref, p_ref, wih_ref, whh_ref, hc_ref, bih_ref, bhh_ref,
             v_ref, zc_s, gi_s, gh_s):
    i = pl.program_id(0)

    @pl.when(i == 0)
    def _front():
        p = p_ref[0, :]
        pn = p / (jnp.sqrt(jnp.sum(p * p)) + 1e-8)
        y = jnp.dot(x_ref[...], pn[:, None],
                    preferred_element_type=jnp.float32)
        _topk_z(x_ref, y, zc_s, N, TK, D)

    @pl.when((i >= 1) & (i <= NGRU))
    def _gru_step():
        gi = jnp.dot(wih_ref[...], zc_s[...],
                     preferred_element_type=jnp.float32)
        gh = jnp.dot(whh_ref[...], hc_ref[...],
                     preferred_element_type=jnp.float32)
        gi_s[pl.ds((i - 1) * RB, RB), :] = gi
        gh_s[pl.ds((i - 1) * RB, RB), :] = gh

    @pl.when(i == NGRU)
    def _gate():
        v_ref[...] = _gates(gi_s[...], gh_s[...], bih_ref[...], bhh_ref[...],
                            hc_ref[...], H0)


def _ka(xp, p2, wih, whh, hcol, bihc, bhhc):
    nsteps = 1 + NGRU
    return pl.pallas_call(
        _ka_body,
        grid=(nsteps,),
        in_specs=[
            pl.BlockSpec((NPAD, D), lambda i: (0, 0)),
            pl.BlockSpec((1, D), lambda i: (0, 0)),
            pl.BlockSpec((RB, TK * D), lambda i: (jnp.clip(i - 1, 0, 5), 0)),
            pl.BlockSpec((RB, H0), lambda i: (jnp.clip(i - 1, 0, 5), 0)),
            pl.BlockSpec((H0, 1), lambda i: (0, 0)),
            pl.BlockSpec((3 * H0, 1), lambda i: (0, 0)),
            pl.BlockSpec((3 * H0, 1), lambda i: (0, 0)),
        ],
        out_specs=pl.BlockSpec((H0, 1), lambda i: (0, 0)),
        out_shape=jax.ShapeDtypeStruct((H0, 1), jnp.float32),
        scratch_shapes=[
            pltpu.VMEM((TK * D, 1), jnp.float32),
            pltpu.VMEM((3 * H0, 1), jnp.float32),
            pltpu.VMEM((3 * H0, 1), jnp.float32),
        ],
    )(xp, p2, wih, whh, hcol, bihc, bhhc)


def _xw_body(x_ref, wnt_ref, degt_ref, yw_ref):
    dt = degt_ref[...]
    dinv = lax.rsqrt(dt[:, 0] + dt[:, 1] + 1.0)[:, None]
    yw_ref[...] = dinv * jnp.dot(x_ref[...], wnt_ref[...],
                                 preferred_element_type=jnp.float32)


def _xw(xp, wnt, degt):
    d = xp.shape[1]
    h = wnt.shape[1]
    return pl.pallas_call(
        _xw_body,
        grid=(NB,),
        in_specs=[
            pl.BlockSpec((BN, d), lambda i: (i, 0)),
            pl.BlockSpec((d, h), lambda i: (0, 0)),
            pl.BlockSpec((BN, 2), lambda i: (i, 0)),
        ],
        out_specs=pl.BlockSpec((BN, h), lambda i: (i, 0)),
        out_shape=jax.ShapeDtypeStruct((NPAD, h), jnp.float32),
    )(xp, wnt, degt)


def _kb_body(agg_ref, yw_ref, degt_ref, bn_ref, g_ref, b_ref, p_ref,
             w1i_ref, w1h_ref, hc1_ref, b1i_ref, b1h_ref,
             yw1_ref, v1_ref, h1_s, y1_s, zc_s):
    i = pl.program_id(0)

    @pl.when(i < NB)
    def _post0():
        dinv = _dinv_col(degt_ref, i)
        o = dinv * (agg_ref[0] + agg_ref[1] + yw_ref[...]) \
            + bn_ref[0, :][None, :]
        mu = jnp.mean(o, axis=-1, keepdims=True)
        var = jnp.mean((o - mu) ** 2, axis=-1, keepdims=True)
        o = (o - mu) * lax.rsqrt(var + 1e-5) * g_ref[0, :][None, :] \
            + b_ref[0, :][None, :]
        o = jnp.maximum(o, 0.0)
        h1_s[pl.ds(i * BN, BN), :] = o
        p = p_ref[0, :]
        pn = p / (jnp.sqrt(jnp.sum(p * p)) + 1e-8)
        y1_s[pl.ds(i * BN, BN), :] = jnp.dot(
            o, pn[:, None], preferred_element_type=jnp.float32)

    @pl.when(i == NB)
    def _mid():
        _topk_z(h1_s, y1_s[...], zc_s, N, TK, HID)
        gi = jnp.dot(w1i_ref[...], zc_s[...],
                     preferred_element_type=jnp.float32)
        gh = jnp.dot(w1h_ref[...], hc1_ref[...],
                     preferred_element_type=jnp.float32)
        v1_ref[...] = _gates(gi, gh, b1i_ref[...], b1h_ref[...],
                             hc1_ref[...], H1)

    @pl.when(i > NB)
    def _xw1():
        j = i - NB - 1
        hb = h1_s[pl.ds(j * BN, BN), :]
        wn = v1_ref[pl.ds(0, OUT * HID), :].reshape(OUT, HID)
        yw1_ref[...] = _dinv_col(degt_ref, j) * lax.dot_general(
            hb, wn, (((1,), (1,)), ((), ())),
            preferred_element_type=jnp.float32)


def _kb(agg0, yw0, degt, bn0, g2, b2, p2, w1i, w1h, hc1, b1i, b1h):
    nsteps = NB + 1 + NB
    return pl.pallas_call(
        _kb_body,
        grid=(nsteps,),
        in_specs=[
            pl.BlockSpec((2, BN, HID), lambda i: (0, jnp.clip(i, 0, 4), 0)),
            pl.BlockSpec((BN, HID), lambda i: (jnp.clip(i, 0, 4), 0)),
            pl.BlockSpec((NPAD, 2), lambda i: (0, 0)),
            pl.BlockSpec((1, HID), lambda i: (0, 0)),
            pl.BlockSpec((1, HID), lambda i: (0, 0)),
            pl.BlockSpec((1, HID), lambda i: (0, 0)),
            pl.BlockSpec((1, HID), lambda i: (0, 0)),
            pl.BlockSpec((3 * H1, TK * HID), lambda i: (0, 0)),
            pl.BlockSpec((3 * H1, H1), lambda i: (0, 0)),
            pl.BlockSpec((H1, 1), lambda i: (0, 0)),
            pl.BlockSpec((3 * H1, 1), lambda i: (0, 0)),
            pl.BlockSpec((3 * H1, 1), lambda i: (0, 0)),
        ],
        out_specs=[
            pl.BlockSpec((BN, HID), lambda i: (jnp.clip(i - 6, 0, 4), 0)),
            pl.BlockSpec((H1, 1), lambda i: (0, 0)),
        ],
        out_shape=[
            jax.ShapeDtypeStruct((NPAD, HID), jnp.float32),
            jax.ShapeDtypeStruct((H1, 1), jnp.float32),
        ],
        scratch_shapes=[
            pltpu.VMEM((NPAD, HID), jnp.float32),
            pltpu.VMEM((NPAD, 1), jnp.float32),
            pltpu.VMEM((TK * HID, 1), jnp.float32),
        ],
    )(agg0, yw0, degt, bn0, g2, b2, p2, w1i, w1h, hc1, b1i, b1h)


def _post_body(agg_ref, yw_ref, degt_ref, bn_ref, out_ref):
    i = pl.program_id(0)
    dinv = _dinv_col(degt_ref, i)
    out_ref[...] = dinv * (agg_ref[0] + agg_ref[1] + yw_ref[...]) \
        + bn_ref[0, :][None, :]


def _post(agg, yw, degt, bn2):
    h = yw.shape[1]
    return pl.pallas_call(
        _post_body,
        grid=(NB,),
        in_specs=[
            pl.BlockSpec((2, BN, h), lambda i: (0, i, 0)),
            pl.BlockSpec((BN, h), lambda i: (i, 0)),
            pl.BlockSpec((NPAD, 2), lambda i: (0, 0)),
            pl.BlockSpec((1, h), lambda i: (0, 0)),
        ],
        out_specs=pl.BlockSpec((BN, h), lambda i: (i, 0)),
        out_shape=jax.ShapeDtypeStruct((NPAD, h), jnp.float32),
    )(agg, yw, degt, bn2)


def kernel(x, edge_index, W0, b0, W1, b1, g0wih, g0whh, g0bih, g0bhh,
           g1wih, g1whh, g1bih, g1bhh, p0, p1, ln_g, ln_b):
    src = edge_index[0]
    dst = edge_index[1]
    padi = jnp.full((EPAD - E,), N, jnp.int32)
    src3 = jnp.concatenate([src, padi]).reshape(NS, TOT, CH)
    dst3 = jnp.concatenate([dst, padi]).reshape(NS, TOT, CH)
    zeros1 = jnp.zeros((NPAD,), jnp.float32)
    zeros2 = jnp.zeros((NPAD, HID), jnp.float32)

    deg = _sc_deg(dst3, zeros1)
    degt = deg.T
    xpad = jnp.concatenate(
        [x, jnp.zeros((NPAD - N, D), jnp.float32)], axis=0)

    hid0 = jnp.concatenate([W0.reshape(-1), b0])
    v0 = _ka(xpad, p0.reshape(1, D), g0wih, g0whh, hid0[:, None],
             g0bih[:, None], g0bhh[:, None])
    wnt0 = v0[:HID * D, 0].reshape(HID, D).T
    yw0 = _xw(xpad, wnt0, degt)
    agg0 = _sc_agg(yw0, src3, dst3, zeros2)

    hid1 = jnp.concatenate([W1.reshape(-1), b1])
    bn0 = v0[HID * D:, 0].reshape(1, HID)
    yw1, v1 = _kb(agg0, yw0, degt, bn0, ln_g.reshape(1, HID),
                  ln_b.reshape(1, HID), p1.reshape(1, HID),
                  g1wih, g1whh, hid1[:, None],
                  g1bih[:, None], g1bhh[:, None])
    agg1 = _sc_agg(yw1, src3, dst3, zeros2)

    bn1 = v1[OUT * HID:, 0].reshape(1, OUT)
    h2 = _post(agg1, yw1, degt, bn1)
    return h2[:N]


# speedup vs baseline: 1.0168x; 1.0168x over previous
import functools

import jax
import jax.numpy as jnp
from jax import lax
from jax.experimental import pallas as pl
from jax.experimental.pallas import tpu as pltpu
from jax.experimental.pallas import tpu_sc as plsc

N = 10000
D = 128
HID = 16
OUT = 16
TK = 8
E = 320000
H0 = HID * D + HID
H1 = OUT * HID + OUT
NC, NS, CH = 2, 16, 128
GB = 8
GROUPS = 20
G0 = 14
TOT = GROUPS * GB
EPAD = NS * TOT * CH
NPAD = 10240
SLAB = NPAD // NS


def _sc_mesh():
    return plsc.VectorSubcoreMesh(core_axis_name="c", subcore_axis_name="s")


_SC_PARAMS = pltpu.CompilerParams(use_tc_tiling_on_sc=False)


def _deg_body(dst_hbm, zeros_hbm, out_hbm, idx_v, ones_v, deg_sh, sem):
    c = lax.axis_index("c")
    s = lax.axis_index("s")
    for j in range(CH // 16):
        ones_v[pl.ds(j * 16, 16)] = jnp.ones((16,), jnp.float32)
    pltpu.sync_copy(zeros_hbm.at[pl.ds(s * SLAB, SLAB)],
                    deg_sh.at[pl.ds(s * SLAB, SLAB)])
    pltpu.sync_copy(dst_hbm.at[s], idx_v)
    plsc.subcore_barrier()
    start = jnp.where(c == 0, 0, G0 * GB)
    ngroups = jnp.where(c == 0, G0, GROUPS - G0)

    def group(g, carry):
        base = start + g * GB
        cps = [pltpu.async_copy(ones_v, deg_sh.at[idx_v.at[base + b]], sem,
                                add=True)
               for b in range(GB)]
        for cp in cps:
            cp.wait()
        return carry

    lax.fori_loop(0, ngroups, group, 0)
    plsc.subcore_barrier()
    pltpu.sync_copy(deg_sh.at[pl.ds(s * SLAB, SLAB)],
                    out_hbm.at[c, pl.ds(s * SLAB, SLAB)])


def _sc_deg(dst3, zeros1):
    return pl.kernel(
        _deg_body,
        out_type=jax.ShapeDtypeStruct((NC, NPAD), jnp.float32),
        mesh=_sc_mesh(),
        compiler_params=_SC_PARAMS,
        scratch_types=[
            pltpu.VMEM((TOT, CH), jnp.int32),
            pltpu.VMEM((CH,), jnp.float32),
            pltpu.VMEM_SHARED((NPAD,), jnp.float32),
            pltpu.SemaphoreType.DMA,
        ],
    )(dst3, zeros1)


def _agg_body(yw_hbm, src_hbm, dst_hbm, zeros_hbm, out_hbm,
              src_v, dst_v, r_a, r_b, acc_sh, yw_sh, gs_a, gs_b, ss_a, ss_b):
    c = lax.axis_index("c")
    s = lax.axis_index("s")
    pltpu.sync_copy(zeros_hbm.at[pl.ds(s * SLAB, SLAB)],
                    acc_sh.at[pl.ds(s * SLAB, SLAB)])
    pltpu.sync_copy(yw_hbm.at[pl.ds(s * SLAB, SLAB)],
                    yw_sh.at[pl.ds(s * SLAB, SLAB)])
    pltpu.sync_copy(src_hbm.at[s], src_v)
    pltpu.sync_copy(dst_hbm.at[s], dst_v)
    plsc.subcore_barrier()
    start = jnp.where(c == 0, 0, G0 * GB)
    ngroups = jnp.where(c == 0, G0, GROUPS - G0)

    def group2(g, carry):
        base = start + g * (2 * GB)
        ga = [pltpu.async_copy(yw_sh.at[src_v.at[base + b]],
                               r_a.at[pl.ds(b * CH, CH)], gs_a)
              for b in range(GB)]
        gb = [pltpu.async_copy(yw_sh.at[src_v.at[base + GB + b]],
                               r_b.at[pl.ds(b * CH, CH)], gs_b)
              for b in range(GB)]
        for cp in ga:
            cp.wait()
        sa = [pltpu.async_copy(r_a.at[pl.ds(b * CH, CH)],
                               acc_sh.at[dst_v.at[base + b]], ss_a, add=True)
              for b in range(GB)]
        for cp in gb:
            cp.wait()
        sb = [pltpu.async_copy(r_b.at[pl.ds(b * CH, CH)],
                               acc_sh.at[dst_v.at[base + GB + b]], ss_b,
                               add=True)
              for b in range(GB)]
        for cp in sa:
            cp.wait()
        for cp in sb:
            cp.wait()
        return carry

    lax.fori_loop(0, ngroups // 2, group2, 0)
    plsc.subcore_barrier()
    pltpu.sync_copy(acc_sh.at[pl.ds(s * SLAB, SLAB)],
                    out_hbm.at[c, pl.ds(s * SLAB, SLAB)])


def _sc_agg(yw, src3, dst3, zeros2):
    return pl.kernel(
        _agg_body,
        out_type=jax.ShapeDtypeStruct((NC, NPAD, HID), jnp.float32),
        mesh=_sc_mesh(),
        compiler_params=_SC_PARAMS,
        scratch_types=[
            pltpu.VMEM((TOT, CH), jnp.int32),
            pltpu.VMEM((TOT, CH), jnp.int32),
            pltpu.VMEM((GB * CH, HID), jnp.float32),
            pltpu.VMEM((GB * CH, HID), jnp.float32),
            pltpu.VMEM_SHARED((NPAD, HID), jnp.float32),
            pltpu.VMEM_SHARED((NPAD, HID), jnp.float32),
            pltpu.SemaphoreType.DMA,
            pltpu.SemaphoreType.DMA,
            pltpu.SemaphoreType.DMA,
            pltpu.SemaphoreType.DMA,
        ],
    )(yw, src3, dst3, zeros2)


BN = 2048
NB = NPAD // BN
RB = 1032
NGRU = 3 * H0 // RB


def _topk_z(h_s, y, zc_s, nvalid, k, d):
    y2 = y.reshape(NPAD // 128, 128)
    gidx = lax.broadcasted_iota(jnp.int32, y2.shape, 0) * 128 \
        + lax.broadcasted_iota(jnp.int32, y2.shape, 1)
    y2 = jnp.where(gidx < nvalid, y2, -jnp.inf)
    for j in range(k):
        mx = jnp.max(y2)
        idx = jnp.min(jnp.where(y2 == mx, gidx, nvalid))
        w = jnp.tanh(mx)
        row = h_s[pl.ds(idx, 1), :]
        zc_s[pl.ds(j * d, d), :] = (row * w).T
        y2 = jnp.where(gidx == idx, -jnp.inf, y2)


def _gates(gi, gh, bih, bhh, h0, h):
    r = jax.nn.sigmoid(gi[0:h, :] + bih[0:h, :]
                       + gh[0:h, :] + bhh[0:h, :])
    z = jax.nn.sigmoid(gi[h:2 * h, :] + bih[h:2 * h, :]
                       + gh[h:2 * h, :] + bhh[h:2 * h, :])
    n = jnp.tanh(gi[2 * h:3 * h, :] + bih[2 * h:3 * h, :]
                 + r * (gh[2 * h:3 * h, :] + bhh[2 * h:3 * h, :]))
    return (1.0 - z) * n + z * h0


def _dinv_col(degt_ref, j):
    dt = degt_ref[pl.ds(j * BN, BN), :]
    return lax.rsqrt(dt[:, 0] + dt[:, 1] + 1.0)[:, None]


def _ka_body(x_ref, p_ref, wih_ref, whh_ref, hc_ref, bih_ref, bhh_ref,
             degt_ref, yw_ref, v_ref, zc_s, gi_s, gh_s):
    i = pl.program_id(0)

    @pl.when(i == 0)
    def _front():
        p = p_ref[0, :]
        pn = p / (jnp.sqrt(jnp.sum(p * p)) + 1e-8)
        y = jnp.dot(x_ref[...], pn[:, None],
                    preferred_element_type=jnp.float32)
        _topk_z(x_ref, y, zc_s, N, TK, D)

    @pl.when((i >= 1) & (i <= NGRU))
    def _gru_step():
        gi = jnp.dot(wih_ref[...], zc_s[...],
                     preferred_element_type=jnp.float32)
        gh = jnp.dot(whh_ref[...], hc_ref[...],
                     preferred_element_type=jnp.float32)
        gi_s[pl.ds((i - 1) * RB, RB), :] = gi
        gh_s[pl.ds((i - 1) * RB, RB), :] = gh

    @pl.when(i == NGRU)
    def _gate():
        v_ref[...] = _gates(gi_s[...], gh_s[...], bih_ref[...], bhh_ref[...],
                            hc_ref[...], H0)

    @pl.when(i > NGRU)
    def _xw():
        j = i - NGRU - 1
        xb = x_ref[pl.ds(j * BN, BN), :]
        wn = v_ref[pl.ds(0, HID * D), :].reshape(HID, D)
        yw_ref[...] = _dinv_col(degt_ref, j) * lax.dot_general(
            xb, wn, (((1,), (1,)), ((), ())),
            preferred_element_type=jnp.float32)


def _ka(xp, p2, wih, whh, hcol, bihc, bhhc, degt):
    nsteps = 1 + NGRU + NB
    return pl.pallas_call(
        _ka_body,
        grid=(nsteps,),
        in_specs=[
            pl.BlockSpec((NPAD, D), lambda i: (0, 0)),
            pl.BlockSpec((1, D), lambda i: (0, 0)),
            pl.BlockSpec((RB, TK * D), lambda i: (jnp.clip(i - 1, 0, 5), 0)),
            pl.BlockSpec((RB, H0), lambda i: (jnp.clip(i - 1, 0, 5), 0)),
            pl.BlockSpec((H0, 1), lambda i: (0, 0)),
            pl.BlockSpec((3 * H0, 1), lambda i: (0, 0)),
            pl.BlockSpec((3 * H0, 1), lambda i: (0, 0)),
            pl.BlockSpec((NPAD, 2), lambda i: (0, 0)),
        ],
        out_specs=[
            pl.BlockSpec((BN, HID), lambda i: (jnp.clip(i - 7, 0, 4), 0)),
            pl.BlockSpec((H0, 1), lambda i: (0, 0)),
        ],
        out_shape=[
            jax.ShapeDtypeStruct((NPAD, HID), jnp.float32),
            jax.ShapeDtypeStruct((H0, 1), jnp.float32),
        ],
        scratch_shapes=[
            pltpu.VMEM((TK * D, 1), jnp.float32),
            pltpu.VMEM((3 * H0, 1), jnp.float32),
            pltpu.VMEM((3 * H0, 1), jnp.float32),
        ],
    )(xp, p2, wih, whh, hcol, bihc, bhhc, degt)


def _kb_body(agg_ref, yw_ref, degt_ref, bn_ref, g_ref, b_ref, p_ref,
             w1i_ref, w1h_ref, hc1_ref, b1i_ref, b1h_ref,
             yw1_ref, v1_ref, h1_s, y1_s, zc_s):
    i = pl.program_id(0)

    @pl.when(i < NB)
    def _post0():
        dinv = _dinv_col(degt_ref, i)
        o = dinv * (agg_ref[0] + agg_ref[1] + yw_ref[...]) \
            + bn_ref[0, :][None, :]
        mu = jnp.mean(o, axis=-1, keepdims=True)
        var = jnp.mean((o - mu) ** 2, axis=-1, keepdims=True)
        o = (o - mu) * lax.rsqrt(var + 1e-5) * g_ref[0, :][None, :] \
            + b_ref[0, :][None, :]
        o = jnp.maximum(o, 0.0)
        h1_s[pl.ds(i * BN, BN), :] = o
        p = p_ref[0, :]
        pn = p / (jnp.sqrt(jnp.sum(p * p)) + 1e-8)
        y1_s[pl.ds(i * BN, BN), :] = jnp.dot(
            o, pn[:, None], preferred_element_type=jnp.float32)

    @pl.when(i == NB)
    def _mid():
        _topk_z(h1_s, y1_s[...], zc_s, N, TK, HID)
        gi = jnp.dot(w1i_ref[...], zc_s[...],
                     preferred_element_type=jnp.float32)
        gh = jnp.dot(w1h_ref[...], hc1_ref[...],
                     preferred_element_type=jnp.float32)
        v1_ref[...] = _gates(gi, gh, b1i_ref[...], b1h_ref[...],
                             hc1_ref[...], H1)

    @pl.when(i > NB)
    def _xw1():
        j = i - NB - 1
        hb = h1_s[pl.ds(j * BN, BN), :]
        wn = v1_ref[pl.ds(0, OUT * HID), :].reshape(OUT, HID)
        yw1_ref[...] = _dinv_col(degt_ref, j) * lax.dot_general(
            hb, wn, (((1,), (1,)), ((), ())),
            preferred_element_type=jnp.float32)


def _kb(agg0, yw0, degt, bn0, g2, b2, p2, w1i, w1h, hc1, b1i, b1h):
    nsteps = NB + 1 + NB
    return pl.pallas_call(
        _kb_body,
        grid=(nsteps,),
        in_specs=[
            pl.BlockSpec((2, BN, HID), lambda i: (0, jnp.clip(i, 0, 4), 0)),
            pl.BlockSpec((BN, HID), lambda i: (jnp.clip(i, 0, 4), 0)),
            pl.BlockSpec((NPAD, 2), lambda i: (0, 0)),
            pl.BlockSpec((1, HID), lambda i: (0, 0)),
            pl.BlockSpec((1, HID), lambda i: (0, 0)),
            pl.BlockSpec((1, HID), lambda i: (0, 0)),
            pl.BlockSpec((1, HID), lambda i: (0, 0)),
            pl.BlockSpec((3 * H1, TK * HID), lambda i: (0, 0)),
            pl.BlockSpec((3 * H1, H1), lambda i: (0, 0)),
            pl.BlockSpec((H1, 1), lambda i: (0, 0)),
            pl.BlockSpec((3 * H1, 1), lambda i: (0, 0)),
            pl.BlockSpec((3 * H1, 1), lambda i: (0, 0)),
        ],
        out_specs=[
            pl.BlockSpec((BN, HID), lambda i: (jnp.clip(i - 6, 0, 4), 0)),
            pl.BlockSpec((H1, 1), lambda i: (0, 0)),
        ],
        out_shape=[
            jax.ShapeDtypeStruct((NPAD, HID), jnp.float32),
            jax.ShapeDtypeStruct((H1, 1), jnp.float32),
        ],
        scratch_shapes=[
            pltpu.VMEM((NPAD, HID), jnp.float32),
            pltpu.VMEM((NPAD, 1), jnp.float32),
            pltpu.VMEM((TK * HID, 1), jnp.float32),
        ],
    )(agg0, yw0, degt, bn0, g2, b2, p2, w1i, w1h, hc1, b1i, b1h)


def _post_body(agg_ref, yw_ref, degt_ref, bn_ref, out_ref):
    i = pl.program_id(0)
    dinv = _dinv_col(degt_ref, i)
    out_ref[...] = dinv * (agg_ref[0] + agg_ref[1] + yw_ref[...]) \
        + bn_ref[0, :][None, :]


def _post(agg, yw, degt, bn2):
    h = yw.shape[1]
    return pl.pallas_call(
        _post_body,
        grid=(NB,),
        in_specs=[
            pl.BlockSpec((2, BN, h), lambda i: (0, i, 0)),
            pl.BlockSpec((BN, h), lambda i: (i, 0)),
            pl.BlockSpec((NPAD, 2), lambda i: (0, 0)),
            pl.BlockSpec((1, h), lambda i: (0, 0)),
        ],
        out_specs=pl.BlockSpec((BN, h), lambda i: (i, 0)),
        out_shape=jax.ShapeDtypeStruct((NPAD, h), jnp.float32),
    )(agg, yw, degt, bn2)


def kernel(x, edge_index, W0, b0, W1, b1, g0wih, g0whh, g0bih, g0bhh,
           g1wih, g1whh, g1bih, g1bhh, p0, p1, ln_g, ln_b):
    src = edge_index[0]
    dst = edge_index[1]
    padi = jnp.full((EPAD - E,), N, jnp.int32)
    src3 = jnp.concatenate([src, padi]).reshape(NS, TOT, CH)
    dst3 = jnp.concatenate([dst, padi]).reshape(NS, TOT, CH)
    zeros1 = jnp.zeros((NPAD,), jnp.float32)
    zeros2 = jnp.zeros((NPAD, HID), jnp.float32)

    deg = _sc_deg(dst3, zeros1)
    degt = deg.T
    xpad = jnp.concatenate(
        [x, jnp.zeros((NPAD - N, D), jnp.float32)], axis=0)

    hid0 = jnp.concatenate([W0.reshape(-1), b0])
    yw0, v0 = _ka(xpad, p0.reshape(1, D), g0wih, g0whh, hid0[:, None],
                  g0bih[:, None], g0bhh[:, None], degt)
    agg0 = _sc_agg(yw0, src3, dst3, zeros2)

    hid1 = jnp.concatenate([W1.reshape(-1), b1])
    bn0 = v0[HID * D:, 0].reshape(1, HID)
    yw1, v1 = _kb(agg0, yw0, degt, bn0, ln_g.reshape(1, HID),
                  ln_b.reshape(1, HID), p1.reshape(1, HID),
                  g1wih, g1whh, hid1[:, None],
                  g1bih[:, None], g1bhh[:, None])
    agg1 = _sc_agg(yw1, src3, dst3, zeros2)

    bn1 = v1[OUT * HID:, 0].reshape(1, OUT)
    h2 = _post(agg1, yw1, degt, bn1)
    return h2[:N]
